# baseline (speedup 1.0000x reference)
import jax
import jax.numpy as jnp
from jax.experimental import pallas as pl
from jax.experimental.pallas import tpu as pltpu

_N, _D = 1_000_000, 3
_HEAD = 8
_CH = 131072
_TAIL = _N - _HEAD
_NCH = -(-_TAIL // _CH)


def _dma_body(x_ref, o_ref, patch, sem_p, sem_b):
    pltpu.make_async_copy(x_ref.at[pl.ds(0, _HEAD)], patch, sem_p).start()

    for i in range(_NCH):
        base = _HEAD + i * _CH
        rows = min(_CH, _N - base)
        pltpu.make_async_copy(
            x_ref.at[pl.ds(base, rows)], o_ref.at[pl.ds(base, rows)], sem_b.at[i]
        ).start()

    pltpu.make_async_copy(x_ref.at[pl.ds(0, _HEAD)], patch, sem_p).wait()
    r = jax.lax.broadcasted_iota(jnp.int32, (_HEAD, _D), 0)
    c = jax.lax.broadcasted_iota(jnp.int32, (_HEAD, _D), 1).astype(jnp.float32)
    vals = patch[...]
    patch[...] = jnp.where(r == 0, 10.0 + c, jnp.where(r == 2, 20.0 + c, vals))
    pltpu.make_async_copy(patch, o_ref.at[pl.ds(0, _HEAD)], sem_p).start()

    for i in range(_NCH):
        base = _HEAD + i * _CH
        rows = min(_CH, _N - base)
        pltpu.make_async_copy(
            x_ref.at[pl.ds(base, rows)], o_ref.at[pl.ds(base, rows)], sem_b.at[i]
        ).wait()
    pltpu.make_async_copy(patch, o_ref.at[pl.ds(0, _HEAD)], sem_p).wait()


def kernel(x):
    return pl.pallas_call(
        _dma_body,
        in_specs=[pl.BlockSpec(memory_space=pl.ANY)],
        out_specs=pl.BlockSpec(memory_space=pl.ANY),
        out_shape=jax.ShapeDtypeStruct((_N, _D), jnp.float32),
        scratch_shapes=[
            pltpu.VMEM((_HEAD, _D), jnp.float32),
            pltpu.SemaphoreType.DMA,
            pltpu.SemaphoreType.DMA((_NCH,)),
        ],
    )(x)


# speedup vs baseline: 4.4370x; 4.4370x over previous
import jax
import jax.numpy as jnp
from jax.experimental import pallas as pl

_N, _D = 1_000_000, 3
_R, _C = 3000, 1000
_BR = 600


def _copy_body(x_ref, o_ref):
    pid = pl.program_id(0)
    vals = x_ref[...]

    @pl.when(pid == 0)
    def _():
        r = jax.lax.broadcasted_iota(jnp.int32, (_BR, _C), 0)
        c = jax.lax.broadcasted_iota(jnp.int32, (_BR, _C), 1)
        cf = c.astype(jnp.float32)
        hit0 = (r == 0) & (c < 3)
        hit2 = (r == 0) & (c >= 6) & (c < 9)
        o_ref[...] = jnp.where(hit0, 10.0 + cf, jnp.where(hit2, 14.0 + cf, vals))

    @pl.when(pid != 0)
    def _():
        o_ref[...] = vals


def kernel(x):
    xv = jnp.reshape(x, (_R, _C))
    out = pl.pallas_call(
        _copy_body,
        grid=(_R // _BR,),
        in_specs=[pl.BlockSpec((_BR, _C), lambda i: (i, 0))],
        out_specs=pl.BlockSpec((_BR, _C), lambda i: (i, 0)),
        out_shape=jax.ShapeDtypeStruct((_R, _C), jnp.float32),
    )(xv)
    return jnp.reshape(out, (_N, _D))
